# baseline (speedup 1.0000x reference)
import functools

import jax
import jax.numpy as jnp
from jax import lax
from jax.experimental import pallas as pl
from jax.experimental.pallas import tpu as pltpu
from jax.experimental.pallas import tpu_sc as plsc

EMBED = 64
LANES = 16
NC = 2
NS = 16
NW = NC * NS

BLK = 1024
G = BLK // 128
HALF = BLK // 2
GH = G // 2


def _build(ntok):
    rows_per_w = ntok // NW
    nblk = rows_per_w // BLK
    mesh = plsc.VectorSubcoreMesh(core_axis_name="c", subcore_axis_name="s")

    @functools.partial(
        pl.kernel,
        out_type=jax.ShapeDtypeStruct((ntok, EMBED), jnp.float32),
        mesh=mesh,
        compiler_params=pltpu.CompilerParams(use_tc_tiling_on_sc=False),
        scratch_types=[
            pltpu.VMEM((G, 128), jnp.int32),
            pltpu.VMEM((G, 128), jnp.int32),
            pltpu.VMEM((HALF, EMBED), jnp.float32),
            pltpu.VMEM((HALF, EMBED), jnp.float32),
            pltpu.SemaphoreType.DMA,
            pltpu.SemaphoreType.DMA,
        ],
    )
    def emb_kernel(tok_hbm, pidx_hbm, table_hbm, pos_hbm, out_hbm,
                   tok_v, pidx_v, rows_v, pos_rows_v, sem_a, sem_b):
        wid = lax.axis_index("s") * NC + lax.axis_index("c")
        w_base = wid * rows_per_w

        def blk_body(ch, carry):
            base = w_base + ch * BLK
            idx_row0 = pl.multiple_of(base // 128, 8)
            pltpu.sync_copy(tok_hbm.at[pl.ds(idx_row0, G)], tok_v)
            pltpu.sync_copy(pidx_hbm.at[pl.ds(idx_row0, G)], pidx_v)
            for h in range(2):
                copies = []
                for j in range(GH):
                    copies.append(pltpu.async_copy(
                        table_hbm.at[tok_v.at[h * GH + j]],
                        rows_v.at[pl.ds(j * 128, 128)], sem_a))
                    copies.append(pltpu.async_copy(
                        pos_hbm.at[pidx_v.at[h * GH + j]],
                        pos_rows_v.at[pl.ds(j * 128, 128)], sem_b))
                for cp in copies:
                    cp.wait()

                def row_body(r, c2):
                    for k in range(EMBED // LANES):
                        sl = pl.ds(k * LANES, LANES)
                        rows_v[r, sl] = rows_v[r, sl] + pos_rows_v[r, sl]
                    return c2

                lax.fori_loop(0, HALF, row_body, 0, unroll=4)
                pltpu.sync_copy(rows_v, out_hbm.at[pl.ds(base + h * HALF, HALF)])
            return carry

        lax.fori_loop(0, nblk, blk_body, 0)

    return emb_kernel


def kernel(x, padding_mask, table, pos_enc):
    b, s = x.shape
    ntok = b * s
    tok = x.reshape(ntok // 128, 128).astype(jnp.int32)
    s_ids = jnp.arange(s, dtype=jnp.int32)[None, :]
    pidx = jnp.where(padding_mask, jnp.int32(s), s_ids)
    pidx = pidx.reshape(ntok // 128, 128).astype(jnp.int32)
    pos_ext = jnp.concatenate(
        [pos_enc.astype(jnp.float32),
         jnp.zeros((1, pos_enc.shape[1]), jnp.float32)], axis=0)
    out = _build(ntok)(tok, pidx, table, pos_ext)
    return out.reshape(b, s, EMBED)
